# Initial kernel scaffold; baseline (speedup 1.0000x reference)
import jax
import jax.numpy as jnp
from jax.experimental import pallas as pl


def _matmul_kernel(x_ref, w_ref, o_ref):
    o_ref[...] = jnp.dot(x_ref[...], w_ref[...],
                         preferred_element_type=jnp.float32)


def _matmul(x, w, block_rows=2000):
    m, k = x.shape
    _, n = w.shape
    grid = (m // block_rows,)
    return pl.pallas_call(
        _matmul_kernel,
        grid=grid,
        in_specs=[
            pl.BlockSpec((block_rows, k), lambda i: (i, 0)),
            pl.BlockSpec((k, n), lambda i: (0, 0)),
        ],
        out_specs=pl.BlockSpec((block_rows, n), lambda i: (i, 0)),
        out_shape=jax.ShapeDtypeStruct((m, n), jnp.float32),
    )(x, w)


def kernel(x, edge_index, W1, a1_src, a1_dst, W2):
    n = x.shape[0]
    src = edge_index[0]
    dst = edge_index[1]

    h1t = _matmul(x, W1)
    al_s = h1t @ a1_src
    al_d = h1t @ a1_dst

    w = jnp.exp(jax.nn.sigmoid(al_s[src] + al_d[dst]))
    denom = jax.ops.segment_sum(w, dst, num_segments=n) + 1e-16

    agg1 = jax.ops.segment_sum(h1t[src] * w[:, None], dst, num_segments=n)
    h1 = jax.nn.elu(agg1 / denom[:, None])
    h2 = _matmul(h1, W2)
    h3t = _matmul(h2, W2.T)
    agg2 = jax.ops.segment_sum(h3t[src] * w[:, None], dst, num_segments=n)
    h3 = jax.nn.elu(agg2 / denom[:, None])
    h4 = _matmul(h3, W1.T)
    return (h2, h4)


# speedup vs baseline: 8.0037x; 8.0037x over previous
import jax
import jax.numpy as jnp
from jax import lax
from jax.experimental import pallas as pl
from jax.experimental.pallas import tpu as pltpu
from jax.experimental.pallas import tpu_sc as plsc

N_NODES = 10000
N_EDGES = 320000
NC = 2
NS = 16
LANES = 16
EB = 128
EPS = N_EDGES // NS
NFB = EPS // EB
TB = EPS - NFB * EB
ROWB = 2000
NRP = 10240

_vmesh = plsc.VectorSubcoreMesh(core_axis_name="c", subcore_axis_name="s")
_sc_params = pltpu.CompilerParams(needs_layout_passes=False)


def _f32(*shape):
    return jax.ShapeDtypeStruct(shape, jnp.float32)


def _edge_w_body(als_hbm, ald_hbm, src_hbm, dst_hbm, w_hbm, parts_hbm,
                 als_v, ald_v, src_v, dst_v, w_v, part_v):
    c = lax.axis_index("c")
    s = lax.axis_index("s")
    wid = c * NS + s
    epw = N_EDGES // (NC * NS)
    base = wid * epw

    pltpu.sync_copy(als_hbm, als_v)
    pltpu.sync_copy(ald_hbm, ald_v)
    pltpu.sync_copy(src_hbm.at[pl.ds(base, epw)], src_v)
    pltpu.sync_copy(dst_hbm.at[pl.ds(base, epw)], dst_v)

    zero16 = jnp.zeros((LANES,), jnp.float32)

    @pl.loop(0, N_NODES, step=LANES)
    def _(i):
        part_v[pl.ds(i, LANES)] = zero16

    @pl.loop(0, epw, step=LANES)
    def _(i):
        s16 = src_v[pl.ds(i, LANES)]
        d16 = dst_v[pl.ds(i, LANES)]
        a = plsc.load_gather(als_v, [s16])
        b = plsc.load_gather(ald_v, [d16])
        e = 1.0 / (1.0 + jnp.exp(-(a + b)))
        w16 = jnp.exp(e)
        w_v[pl.ds(i, LANES)] = w16
        plsc.addupdate_scatter(part_v, [d16], w16)

    pltpu.sync_copy(w_v, w_hbm.at[pl.ds(base, epw)])
    pltpu.sync_copy(part_v, parts_hbm.at[wid])


@jax.jit
def _edge_w(al_s, al_d, src, dst):
    epw = N_EDGES // (NC * NS)
    return pl.kernel(
        _edge_w_body,
        out_type=[_f32(N_EDGES), _f32(NC * NS, N_NODES)],
        mesh=_vmesh,
        compiler_params=_sc_params,
        scratch_types=[
            pltpu.VMEM((N_NODES,), jnp.float32),
            pltpu.VMEM((N_NODES,), jnp.float32),
            pltpu.VMEM((epw,), jnp.int32),
            pltpu.VMEM((epw,), jnp.int32),
            pltpu.VMEM((epw,), jnp.float32),
            pltpu.VMEM((N_NODES,), jnp.float32),
        ],
    )(al_s, al_d, src, dst)


def _agg_body(table3, src_hbm, dst_hbm, w_hbm, zeros_hbm, out_hbm,
              srcb, dstb, wb_v, rows_v, srct, dstt, wt_v, rowst, acc, sem):
    c = lax.axis_index("c")
    s = lax.axis_index("s")
    rows_per_sub = NRP // NS
    base = s * EPS

    pltpu.sync_copy(zeros_hbm.at[pl.ds(s * rows_per_sub, rows_per_sub)],
                    acc.at[pl.ds(s * rows_per_sub, rows_per_sub)])

    plsc.subcore_barrier()

    def batch(off, sidx, didx, w_ref, rows, nb):
        pltpu.sync_copy(src_hbm.at[pl.ds(off, nb)], sidx)
        pltpu.sync_copy(dst_hbm.at[pl.ds(off, nb)], didx)
        pltpu.sync_copy(w_hbm.at[pl.ds(off, nb)], w_ref)
        pltpu.async_copy(table3.at[c].at[sidx], rows, sem).wait()

        @pl.loop(0, nb)
        def _(e):
            wbc = plsc.load_gather(w_ref, [jnp.full((LANES,), e, jnp.int32)])
            for k in range(128 // LANES):
                sl = (e, pl.ds(k * LANES, LANES))
                rows[sl] = rows[sl] * wbc

        pltpu.sync_copy(rows, acc.at[didx], add=True)

    @pl.loop(0, NFB)
    def _(j):
        batch(base + j * EB, srcb, dstb, wb_v, rows_v, EB)

    batch(base + NFB * EB, srct, dstt, wt_v, rowst, TB)

    plsc.subcore_barrier()

    pltpu.sync_copy(acc.at[pl.ds(s * rows_per_sub, rows_per_sub)],
                    out_hbm.at[c].at[pl.ds(s * rows_per_sub, rows_per_sub)])


@jax.jit
def _aggregate(table3, src, dst, w, zeros_hbm):
    return pl.kernel(
        _agg_body,
        out_type=_f32(NC, NRP, 128),
        mesh=_vmesh,
        compiler_params=_sc_params,
        scratch_types=[
            pltpu.VMEM((EB,), jnp.int32),
            pltpu.VMEM((EB,), jnp.int32),
            pltpu.VMEM((EB,), jnp.float32),
            pltpu.VMEM((EB, 128), jnp.float32),
            pltpu.VMEM((TB,), jnp.int32),
            pltpu.VMEM((TB,), jnp.int32),
            pltpu.VMEM((TB,), jnp.float32),
            pltpu.VMEM((TB, 128), jnp.float32),
            pltpu.VMEM_SHARED((NRP, 128), jnp.float32),
            pltpu.SemaphoreType.DMA,
        ],
    )(table3, src, dst, w, zeros_hbm)


def _lin1_kernel(x_ref, w1h_ref, av2_ref, tab_ref, al_ref):
    tab_ref[0] = jnp.dot(x_ref[...], w1h_ref[0],
                         preferred_element_type=jnp.float32)
    al_ref[...] = jnp.dot(x_ref[...], av2_ref[...],
                          preferred_element_type=jnp.float32)


@jax.jit
def _lin1(x, W1h, av2):
    n, k = x.shape
    return pl.pallas_call(
        _lin1_kernel,
        grid=(n // ROWB, 2),
        in_specs=[
            pl.BlockSpec((ROWB, k), lambda i, j: (i, 0)),
            pl.BlockSpec((1, k, 128), lambda i, j: (j, 0, 0)),
            pl.BlockSpec((k, 2), lambda i, j: (0, 0)),
        ],
        out_specs=[
            pl.BlockSpec((1, ROWB, 128), lambda i, j: (j, i, 0)),
            pl.BlockSpec((ROWB, 2), lambda i, j: (i, 0)),
        ],
        out_shape=[_f32(2, n, 128), _f32(n, 2)],
    )(x, W1h, av2)


def _mid_kernel(a0_ref, a1_ref, parts_ref, w2_ref, w2th_ref,
                h2_ref, tab_ref):
    denom = jnp.sum(parts_ref[...], axis=1) + 1e-16
    agg = jnp.concatenate([a0_ref[0], a1_ref[0]], axis=1)
    z = agg / denom[:, None]
    h1 = jnp.where(z > 0, z, jnp.exp(jnp.minimum(z, 0.0)) - 1.0)
    h2 = jnp.dot(h1, w2_ref[...], preferred_element_type=jnp.float32)
    h2_ref[...] = h2
    tab_ref[0] = jnp.dot(h2, w2th_ref[0],
                         preferred_element_type=jnp.float32)


@jax.jit
def _mid(agg, parts_t, W2, W2th):
    hid, out_d = W2.shape
    n = N_NODES
    return pl.pallas_call(
        _mid_kernel,
        grid=(n // ROWB, 2),
        in_specs=[
            pl.BlockSpec((1, ROWB, 128), lambda i, j: (0, i, 0)),
            pl.BlockSpec((1, ROWB, 128), lambda i, j: (1, i, 0)),
            pl.BlockSpec((ROWB, NC * NS), lambda i, j: (i, 0)),
            pl.BlockSpec((hid, out_d), lambda i, j: (0, 0)),
            pl.BlockSpec((1, out_d, 128), lambda i, j: (j, 0, 0)),
        ],
        out_specs=[
            pl.BlockSpec((ROWB, out_d), lambda i, j: (i, 0)),
            pl.BlockSpec((1, ROWB, 128), lambda i, j: (j, i, 0)),
        ],
        out_shape=[_f32(n, out_d), _f32(2, n, 128)],
    )(agg[:, :n], agg[:, :n], parts_t, W2, W2th)


def _out_kernel(a0_ref, a1_ref, parts_ref, w1t_ref, h4_ref):
    denom = jnp.sum(parts_ref[...], axis=1) + 1e-16
    agg = jnp.concatenate([a0_ref[0], a1_ref[0]], axis=1)
    z = agg / denom[:, None]
    h3 = jnp.where(z > 0, z, jnp.exp(jnp.minimum(z, 0.0)) - 1.0)
    h4_ref[...] = jnp.dot(h3, w1t_ref[...], preferred_element_type=jnp.float32)


@jax.jit
def _final(agg, parts_t, W1t):
    hid, in_d = W1t.shape
    n = N_NODES
    return pl.pallas_call(
        _out_kernel,
        grid=(n // ROWB,),
        in_specs=[
            pl.BlockSpec((1, ROWB, 128), lambda i: (0, i, 0)),
            pl.BlockSpec((1, ROWB, 128), lambda i: (1, i, 0)),
            pl.BlockSpec((ROWB, NC * NS), lambda i: (i, 0)),
            pl.BlockSpec((hid, in_d), lambda i: (0, 0)),
        ],
        out_specs=pl.BlockSpec((ROWB, in_d), lambda i: (i, 0)),
        out_shape=_f32(n, in_d),
    )(agg[:, :n], agg[:, :n], parts_t, W1t)


def kernel(x, edge_index, W1, a1_src, a1_dst, W2):
    src = edge_index[0]
    dst = edge_index[1]

    av2 = W1 @ jnp.stack([a1_src, a1_dst], axis=1)
    W1h = W1.reshape(128, 2, 128).transpose(1, 0, 2)
    W2th = W2.T.reshape(64, 2, 128).transpose(1, 0, 2)

    table1, al = _lin1(x, W1h, av2)
    al_s = al[:, 0]
    al_d = al[:, 1]

    w, parts = _edge_w(al_s, al_d, src, dst)
    parts_t = parts.T

    zeros_hbm = jnp.zeros((NRP, 128), jnp.float32)

    agg1 = _aggregate(table1, src, dst, w, zeros_hbm)
    h2, table2 = _mid(agg1, parts_t, W2, W2th)

    agg2 = _aggregate(table2, src, dst, w, zeros_hbm)
    h4 = _final(agg2, parts_t, W1.T)
    return (h2, h4)


# speedup vs baseline: 13.5382x; 1.6915x over previous
import jax
import jax.numpy as jnp
from jax import lax
from jax.experimental import pallas as pl
from jax.experimental.pallas import tpu as pltpu
from jax.experimental.pallas import tpu_sc as plsc

N_NODES = 10000
N_EDGES = 320000
NC = 2
NS = 16
LANES = 16
EB = 128
NB = 158
EPAD = NS * NB * EB
EPS = EPAD // NS
ROWB = 2000
NRP = 10240

_vmesh = plsc.VectorSubcoreMesh(core_axis_name="c", subcore_axis_name="s")
_sc_params = pltpu.CompilerParams(needs_layout_passes=False)


def _f32(*shape):
    return jax.ShapeDtypeStruct(shape, jnp.float32)


def _edge_w_body(als_hbm, ald_hbm, src_hbm, dst_hbm, w_hbm, parts_hbm,
                 als_v, ald_v, src_v, dst_v, w_v, part_v):
    c = lax.axis_index("c")
    s = lax.axis_index("s")
    wid = c * NS + s
    epw = N_EDGES // (NC * NS)
    base = wid * epw

    pltpu.sync_copy(als_hbm, als_v)
    pltpu.sync_copy(ald_hbm, ald_v)
    pltpu.sync_copy(src_hbm.at[pl.ds(base, epw)], src_v)
    pltpu.sync_copy(dst_hbm.at[pl.ds(base, epw)], dst_v)

    zero16 = jnp.zeros((LANES,), jnp.float32)

    @pl.loop(0, N_NODES, step=LANES)
    def _(i):
        part_v[pl.ds(i, LANES)] = zero16

    @pl.loop(0, epw, step=LANES)
    def _(i):
        s16 = src_v[pl.ds(i, LANES)]
        d16 = dst_v[pl.ds(i, LANES)]
        a = plsc.load_gather(als_v, [s16])
        b = plsc.load_gather(ald_v, [d16])
        e = 1.0 / (1.0 + jnp.exp(-(a + b)))
        w16 = jnp.exp(e)
        w_v[pl.ds(i, LANES)] = w16
        plsc.addupdate_scatter(part_v, [d16], w16)

    pltpu.sync_copy(w_v, w_hbm.at[pl.ds(base, epw)])
    pltpu.sync_copy(part_v, parts_hbm.at[wid])


@jax.jit
def _edge_w(al_s, al_d, src, dst):
    epw = N_EDGES // (NC * NS)
    return pl.kernel(
        _edge_w_body,
        out_type=[_f32(N_EDGES), _f32(NC * NS, N_NODES)],
        mesh=_vmesh,
        compiler_params=_sc_params,
        scratch_types=[
            pltpu.VMEM((N_NODES,), jnp.float32),
            pltpu.VMEM((N_NODES,), jnp.float32),
            pltpu.VMEM((epw,), jnp.int32),
            pltpu.VMEM((epw,), jnp.int32),
            pltpu.VMEM((epw,), jnp.float32),
            pltpu.VMEM((N_NODES,), jnp.float32),
        ],
    )(al_s, al_d, src, dst)


def _agg_body(table3, src_hbm, dst_hbm, w_hbm, zeros_hbm, out_hbm,
              sb0, sb1, db0, db1, wb0, wb1, r0, r1, acc, gsem, msem):
    c = lax.axis_index("c")
    s = lax.axis_index("s")
    rows_per_sub = NRP // NS
    base = s * EPS
    S = (sb0, sb1)
    D = (db0, db1)
    W = (wb0, wb1)
    R = (r0, r1)

    pltpu.sync_copy(zeros_hbm.at[pl.ds(s * rows_per_sub, rows_per_sub)],
                    acc.at[pl.ds(s * rows_per_sub, rows_per_sub)])

    plsc.subcore_barrier()

    def fetch_meta(j, p, sem):
        off = base + j * EB
        pltpu.async_copy(src_hbm.at[pl.ds(off, EB)], S[p], sem)
        pltpu.async_copy(dst_hbm.at[pl.ds(off, EB)], D[p], sem)
        pltpu.async_copy(w_hbm.at[pl.ds(off, EB)], W[p], sem)

    def drain_meta(p, sem):
        pltpu.make_async_copy(src_hbm.at[pl.ds(0, EB)], S[p], sem).wait()
        pltpu.make_async_copy(dst_hbm.at[pl.ds(0, EB)], D[p], sem).wait()
        pltpu.make_async_copy(w_hbm.at[pl.ds(0, EB)], W[p], sem).wait()

    def compute(p):
        @pl.loop(0, EB)
        def _(e):
            wbc = plsc.load_gather(W[p], [jnp.full((LANES,), e, jnp.int32)])
            for k in range(128 // LANES):
                sl = (e, pl.ds(k * LANES, LANES))
                R[p][sl] = R[p][sl] * wbc

        pltpu.sync_copy(R[p], acc.at[D[p]], add=True)

    fetch_meta(0, 0, msem)
    drain_meta(0, msem)
    pltpu.async_copy(table3.at[c].at[S[0]], R[0], gsem)
    fetch_meta(1, 1, msem)

    @pl.loop(0, NB // 2 - 1)
    def _(g):
        for p in (0, 1):
            pltpu.make_async_copy(zeros_hbm.at[pl.ds(0, EB)], R[p], gsem).wait()
            drain_meta(1 - p, msem)
            pltpu.async_copy(table3.at[c].at[S[1 - p]], R[1 - p], gsem)
            compute(p)
            fetch_meta(2 * g + p + 2, p, msem)

    pltpu.make_async_copy(zeros_hbm.at[pl.ds(0, EB)], R[0], gsem).wait()
    drain_meta(1, msem)
    pltpu.async_copy(table3.at[c].at[S[1]], R[1], gsem)
    compute(0)
    pltpu.make_async_copy(zeros_hbm.at[pl.ds(0, EB)], R[1], gsem).wait()
    compute(1)

    plsc.subcore_barrier()

    pltpu.sync_copy(acc.at[pl.ds(s * rows_per_sub, rows_per_sub)],
                    out_hbm.at[c].at[pl.ds(s * rows_per_sub, rows_per_sub)])


@jax.jit
def _aggregate(table3, src, dst, w, zeros_hbm):
    return pl.kernel(
        _agg_body,
        out_type=_f32(NC, NRP, 128),
        mesh=_vmesh,
        compiler_params=_sc_params,
        scratch_types=[
            pltpu.VMEM((EB,), jnp.int32),
            pltpu.VMEM((EB,), jnp.int32),
            pltpu.VMEM((EB,), jnp.int32),
            pltpu.VMEM((EB,), jnp.int32),
            pltpu.VMEM((EB,), jnp.float32),
            pltpu.VMEM((EB,), jnp.float32),
            pltpu.VMEM((EB, 128), jnp.float32),
            pltpu.VMEM((EB, 128), jnp.float32),
            pltpu.VMEM_SHARED((NRP, 128), jnp.float32),
            pltpu.SemaphoreType.DMA,
            pltpu.SemaphoreType.DMA,
        ],
    )(table3, src, dst, w, zeros_hbm)


def _lin1_kernel(x_ref, w1h_ref, av2_ref, tab_ref, al_ref):
    tab_ref[0] = jnp.dot(x_ref[...], w1h_ref[0],
                         preferred_element_type=jnp.float32)
    al_ref[...] = jnp.dot(x_ref[...], av2_ref[...],
                          preferred_element_type=jnp.float32)


@jax.jit
def _lin1(x, W1h, av2):
    n, k = x.shape
    return pl.pallas_call(
        _lin1_kernel,
        grid=(n // ROWB, 2),
        in_specs=[
            pl.BlockSpec((ROWB, k), lambda i, j: (i, 0)),
            pl.BlockSpec((1, k, 128), lambda i, j: (j, 0, 0)),
            pl.BlockSpec((k, 2), lambda i, j: (0, 0)),
        ],
        out_specs=[
            pl.BlockSpec((1, ROWB, 128), lambda i, j: (j, i, 0)),
            pl.BlockSpec((ROWB, 2), lambda i, j: (i, 0)),
        ],
        out_shape=[_f32(2, n, 128), _f32(n, 2)],
    )(x, W1h, av2)


def _mid_kernel(a0_ref, a1_ref, parts_ref, w2_ref, w2th_ref,
                h2_ref, tab_ref):
    denom = jnp.sum(parts_ref[...], axis=1) + 1e-16
    agg = jnp.concatenate([a0_ref[0], a1_ref[0]], axis=1)
    z = agg / denom[:, None]
    h1 = jnp.where(z > 0, z, jnp.exp(jnp.minimum(z, 0.0)) - 1.0)
    h2 = jnp.dot(h1, w2_ref[...], preferred_element_type=jnp.float32)
    h2_ref[...] = h2
    tab_ref[0] = jnp.dot(h2, w2th_ref[0],
                         preferred_element_type=jnp.float32)


@jax.jit
def _mid(agg, parts_t, W2, W2th):
    hid, out_d = W2.shape
    n = N_NODES
    return pl.pallas_call(
        _mid_kernel,
        grid=(n // ROWB, 2),
        in_specs=[
            pl.BlockSpec((1, ROWB, 128), lambda i, j: (0, i, 0)),
            pl.BlockSpec((1, ROWB, 128), lambda i, j: (1, i, 0)),
            pl.BlockSpec((ROWB, NC * NS), lambda i, j: (i, 0)),
            pl.BlockSpec((hid, out_d), lambda i, j: (0, 0)),
            pl.BlockSpec((1, out_d, 128), lambda i, j: (j, 0, 0)),
        ],
        out_specs=[
            pl.BlockSpec((ROWB, out_d), lambda i, j: (i, 0)),
            pl.BlockSpec((1, ROWB, 128), lambda i, j: (j, i, 0)),
        ],
        out_shape=[_f32(n, out_d), _f32(2, n, 128)],
    )(agg[:, :n], agg[:, :n], parts_t, W2, W2th)


def _out_kernel(a0_ref, a1_ref, parts_ref, w1t_ref, h4_ref):
    denom = jnp.sum(parts_ref[...], axis=1) + 1e-16
    agg = jnp.concatenate([a0_ref[0], a1_ref[0]], axis=1)
    z = agg / denom[:, None]
    h3 = jnp.where(z > 0, z, jnp.exp(jnp.minimum(z, 0.0)) - 1.0)
    h4_ref[...] = jnp.dot(h3, w1t_ref[...], preferred_element_type=jnp.float32)


@jax.jit
def _final(agg, parts_t, W1t):
    hid, in_d = W1t.shape
    n = N_NODES
    return pl.pallas_call(
        _out_kernel,
        grid=(n // ROWB,),
        in_specs=[
            pl.BlockSpec((1, ROWB, 128), lambda i: (0, i, 0)),
            pl.BlockSpec((1, ROWB, 128), lambda i: (1, i, 0)),
            pl.BlockSpec((ROWB, NC * NS), lambda i: (i, 0)),
            pl.BlockSpec((hid, in_d), lambda i: (0, 0)),
        ],
        out_specs=pl.BlockSpec((ROWB, in_d), lambda i: (i, 0)),
        out_shape=_f32(n, in_d),
    )(agg[:, :n], agg[:, :n], parts_t, W1t)


def kernel(x, edge_index, W1, a1_src, a1_dst, W2):
    src = edge_index[0]
    dst = edge_index[1]

    av2 = W1 @ jnp.stack([a1_src, a1_dst], axis=1)
    W1h = W1.reshape(128, 2, 128).transpose(1, 0, 2)
    W2th = W2.T.reshape(64, 2, 128).transpose(1, 0, 2)

    table1, al = _lin1(x, W1h, av2)
    al_s = al[:, 0]
    al_d = al[:, 1]

    w, parts = _edge_w(al_s, al_d, src, dst)
    parts_t = parts.T

    npad = EPAD - N_EDGES
    pad_idx = jnp.arange(npad, dtype=jnp.int32) % N_NODES
    srcp = jnp.concatenate([src, pad_idx])
    dstp = jnp.concatenate([dst, pad_idx])
    wp = jnp.concatenate([w, jnp.zeros((npad,), jnp.float32)])

    zeros_hbm = jnp.zeros((NRP, 128), jnp.float32)

    agg1 = _aggregate(table1, srcp, dstp, wp, zeros_hbm)
    h2, table2 = _mid(agg1, parts_t, W2, W2th)

    agg2 = _aggregate(table2, srcp, dstp, wp, zeros_hbm)
    h4 = _final(agg2, parts_t, W1.T)
    return (h2, h4)
